# Initial kernel scaffold; baseline (speedup 1.0000x reference)
import jax
import jax.numpy as jnp
from jax.experimental import pallas as pl


def kernel(x, A, B):
    raise NotImplementedError("write your pallas kernel here")


# speedup vs baseline: 15.9392x; 15.9392x over previous
import functools

import jax
import jax.numpy as jnp
from jax import lax
from jax.experimental import pallas as pl
from jax.experimental.pallas import tpu as pltpu
from jax.experimental.pallas import tpu_sc as plsc

NUM_EMB = 1000000
DIM = 32
R = 4
SCALING = 1.0 / 4.0
BATCH = 16384
HIST = 200
NTOK = BATCH * HIST


GROUPS = NUM_EMB // 4
VC4 = 2048


def _table_body(a_ref, w_ref, t_ref):
    t_ref[...] = jnp.dot(a_ref[...], w_ref[...],
                         preferred_element_type=jnp.float32)


def _build_table(At4, W16):
    return pl.pallas_call(
        _table_body,
        grid=((GROUPS + VC4 - 1) // VC4,),
        in_specs=[
            pl.BlockSpec((VC4, 4 * R), lambda i: (i, 0)),
            pl.BlockSpec((4 * R, 4 * DIM), lambda i: (0, 0)),
        ],
        out_specs=pl.BlockSpec((VC4, 4 * DIM), lambda i: (i, 0)),
        out_shape=jax.ShapeDtypeStruct((GROUPS, 4 * DIM), jnp.float32),
    )(At4, W16)


_info = plsc.get_sparse_core_info()
NC, NS = _info.num_cores, _info.num_subcores
NW = NC * NS
TOK_PER_W = NTOK // NW
CHK = 1024
NSTEP = TOK_PER_W // CHK


def _gather_body(t_hbm, x_hbm, out_hbm, idx_v, rows_v, sem):
    wid = lax.axis_index("s") * NC + lax.axis_index("c")
    base = wid * TOK_PER_W

    def step(g, carry):
        off = base + g * CHK
        pltpu.sync_copy(x_hbm.at[pl.ds(off, CHK)], idx_v)
        pltpu.async_copy(t_hbm.at[idx_v], rows_v, sem).wait()
        pltpu.sync_copy(rows_v, out_hbm.at[pl.ds(off, CHK)])
        return carry

    lax.fori_loop(0, NSTEP, step, 0)


@jax.jit
def _run(x_flat, A, B):
    At4 = A.T.reshape(GROUPS, 4 * R)
    W16 = jnp.kron(jnp.eye(4, dtype=jnp.float32), B.T) * SCALING
    t128 = _build_table(At4, W16)
    table = t128.reshape(NUM_EMB, DIM)
    mesh = plsc.VectorSubcoreMesh(core_axis_name="c", subcore_axis_name="s")
    gather = pl.kernel(
        _gather_body,
        out_type=jax.ShapeDtypeStruct((NTOK, DIM), jnp.float32),
        mesh=mesh,
        scratch_types=[
            pltpu.VMEM((CHK,), jnp.int32),
            pltpu.VMEM((CHK, DIM), jnp.float32),
            pltpu.SemaphoreType.DMA,
        ],
        compiler_params=pltpu.CompilerParams(use_tc_tiling_on_sc=False),
    )
    return gather(table, x_flat)


def kernel(x, A, B):
    x_flat = x.reshape(-1).astype(jnp.int32)
    out = _run(x_flat, A, B)
    return out.reshape(BATCH, HIST, DIM)


# speedup vs baseline: 56.0322x; 3.5154x over previous
import jax
import jax.numpy as jnp
from jax import lax
from jax.experimental import pallas as pl
from jax.experimental.pallas import tpu as pltpu
from jax.experimental.pallas import tpu_sc as plsc

NUM_EMB = 1000000
DIM = 32
R = 4
SCALING = 1.0 / 4.0
BATCH = 16384
HIST = 200
NTOK = BATCH * HIST

_info = plsc.get_sparse_core_info()
NC, NS = _info.num_cores, _info.num_subcores
NW = NC * NS

_SC_PARAMS = pltpu.CompilerParams(use_tc_tiling_on_sc=False,
                                  needs_layout_passes=False)


VCH = 8000
NVCH = NUM_EMB // VCH
CPW = (NVCH + NW - 1) // NW


def _pack_body(a_hbm, at_hbm, planes_v, stage_v):
    wid = lax.axis_index("s") * NC + lax.axis_index("c")
    lanes = lax.broadcasted_iota(jnp.int32, (16,), 0)
    r_idx = lanes & 7
    r_clamped = r_idx & 3
    l8 = lanes >> 3
    valid = r_idx < R
    fzero = jnp.zeros((16,), jnp.float32)

    def chunk(i, carry):
        c = wid + i * NW

        @pl.when(c < NVCH)
        def _():
            for r in range(R):
                pltpu.sync_copy(a_hbm.at[pl.ds(r * NUM_EMB + c * VCH, VCH)],
                                planes_v.at[r])

            def g_step(g, cc):
                vals = plsc.load_gather(planes_v, [r_clamped, g * 2 + l8])
                stage_v[pl.ds(g * 16, 16)] = jnp.where(valid, vals, fzero)
                return cc

            lax.fori_loop(0, VCH * 8 // 16, g_step, 0)
            pltpu.sync_copy(stage_v, at_hbm.at[pl.ds(c * VCH * 8, VCH * 8)])

        return carry

    lax.fori_loop(0, CPW, chunk, 0)


BPW = BATCH // NW
STAGE = BPW * 8


def _gather_body(t_hbm, x_hbm, e_hbm, idx_v, rows_v, stage_v, sem):
    wid = lax.axis_index("s") * NC + lax.axis_index("c")
    zero = jnp.zeros((16,), jnp.float32)

    def zstep(i, c):
        stage_v[pl.ds(i * 16, 16)] = zero
        return c

    lax.fori_loop(0, STAGE // 16, zstep, 0)

    lanes = lax.broadcasted_iota(jnp.int32, (16,), 0)

    def step(h, carry):
        off = h * BATCH + wid * BPW
        pltpu.sync_copy(x_hbm.at[pl.ds(off, BPW)], idx_v)
        pltpu.async_copy(t_hbm.at[idx_v], rows_v, sem).wait()

        def dstep(j, c):
            t0 = j * 16
            dst0 = (t0 // 128) * 1024 + t0 % 128
            for r in range(R):
                vals = plsc.load_gather(
                    rows_v, [t0 + lanes, jnp.full((16,), r, jnp.int32)])
                stage_v[pl.ds(dst0 + r * 128, 16)] = vals
            return c

        lax.fori_loop(0, BPW // 16, dstep, 0)
        pltpu.sync_copy(stage_v,
                        e_hbm.at[pl.ds(h * (8 * BATCH) + wid * STAGE, STAGE)])
        return carry

    lax.fori_loop(0, HIST, step, 0)


def _matmul_body(b_ref, e_ref, o_ref):
    b8 = b_ref[...]
    for bt in range(BATCH // 128):
        e = e_ref[pl.ds(bt * 8, 8), :]
        o_ref[0, :, pl.ds(bt * 128, 128)] = jnp.dot(
            b8, e, preferred_element_type=jnp.float32)


@jax.jit
def _run(x, A, B):
    x_flat = x.T.reshape(NTOK).astype(jnp.int32)
    a_flat = A.reshape(R * NUM_EMB)

    mesh = plsc.VectorSubcoreMesh(core_axis_name="c", subcore_axis_name="s")
    pack = pl.kernel(
        _pack_body,
        out_type=jax.ShapeDtypeStruct((8 * NUM_EMB,), jnp.float32),
        mesh=mesh,
        scratch_types=[
            pltpu.VMEM((R, VCH), jnp.float32),
            pltpu.VMEM((VCH * 8,), jnp.float32),
        ],
        compiler_params=_SC_PARAMS,
    )
    at_lin = pack(a_flat)
    at = at_lin.reshape(NUM_EMB, 8)

    gather = pl.kernel(
        _gather_body,
        out_type=jax.ShapeDtypeStruct((HIST * 8 * BATCH,), jnp.float32),
        mesh=mesh,
        scratch_types=[
            pltpu.VMEM((BPW,), jnp.int32),
            pltpu.VMEM((BPW, 8), jnp.float32),
            pltpu.VMEM((STAGE,), jnp.float32),
            pltpu.SemaphoreType.DMA,
        ],
        compiler_params=_SC_PARAMS,
    )
    e_flat = gather(at, x_flat)
    e2 = e_flat.reshape(HIST * 8 * BATCH // 128, 128)

    B8 = jnp.concatenate([B * SCALING, jnp.zeros((DIM, 4), jnp.float32)],
                         axis=1)
    outT = pl.pallas_call(
        _matmul_body,
        grid=(HIST,),
        in_specs=[
            pl.BlockSpec((DIM, 8), lambda i: (0, 0)),
            pl.BlockSpec((BATCH // 16, 128), lambda i: (i, 0)),
        ],
        out_specs=pl.BlockSpec((1, DIM, BATCH), lambda i: (i, 0, 0)),
        out_shape=jax.ShapeDtypeStruct((HIST, DIM, BATCH), jnp.float32),
    )(B8, e2)
    return jnp.transpose(outT, (2, 0, 1))


def kernel(x, A, B):
    return _run(x, A, B)


# speedup vs baseline: 85.6804x; 1.5291x over previous
import jax
import jax.numpy as jnp
from jax import lax
from jax.experimental import pallas as pl
from jax.experimental.pallas import tpu as pltpu
from jax.experimental.pallas import tpu_sc as plsc

NUM_EMB = 1000000
DIM = 32
R = 4
SCALING = 1.0 / 4.0
BATCH = 16384
HIST = 200
NTOK = BATCH * HIST

_info = plsc.get_sparse_core_info()
NC, NS = _info.num_cores, _info.num_subcores
NW = NC * NS

_SC_PARAMS = pltpu.CompilerParams(use_tc_tiling_on_sc=False,
                                  needs_layout_passes=False)


VCH = 8000
NVCH = NUM_EMB // VCH
CPW = (NVCH + NW - 1) // NW


def _pack_body(a_hbm, at_hbm, planes_v, stage_v):
    wid = lax.axis_index("s") * NC + lax.axis_index("c")
    lanes = lax.broadcasted_iota(jnp.int32, (16,), 0)
    r_idx = lanes & 7
    r_clamped = r_idx & 3
    l8 = lanes >> 3
    valid = r_idx < R
    fzero = jnp.zeros((16,), jnp.float32)

    def chunk(i, carry):
        c = wid + i * NW

        @pl.when(c < NVCH)
        def _():
            for r in range(R):
                pltpu.sync_copy(a_hbm.at[pl.ds(r * NUM_EMB + c * VCH, VCH)],
                                planes_v.at[r])

            def g_step(g, cc):
                vals = plsc.load_gather(planes_v, [r_clamped, g * 2 + l8])
                stage_v[pl.ds(g * 16, 16)] = jnp.where(valid, vals, fzero)
                return cc

            lax.fori_loop(0, VCH * 8 // 16, g_step, 0)
            pltpu.sync_copy(stage_v, at_hbm.at[pl.ds(c * VCH * 8, VCH * 8)])

        return carry

    lax.fori_loop(0, CPW, chunk, 0)


NHG = 8
NBG = NW // NHG
HPW = HIST // NHG
BSL = BATCH // NBG
CHK = 2048
SUBS = BSL // CHK
NCH = HPW * SUBS
STAGE = CHK * 8


def _gather_body(t_hbm, x_hbm, e_hbm, idx_v, rows_v, stage_v,
                 gsem0, gsem1, osem0, osem1):
    wid = lax.axis_index("s") * NC + lax.axis_index("c")
    hg = wid >> 2
    bg = wid & 3
    h0 = hg * HPW
    b0 = bg * BSL
    gsems = (gsem0, gsem1)
    osems = (osem0, osem1)
    zero = jnp.zeros((16,), jnp.float32)

    def zstep(i, c):
        stage_v[pl.ds(i * 16, 16)] = zero
        return c

    lax.fori_loop(0, 2 * STAGE // 16, zstep, 0)

    lanes = lax.broadcasted_iota(jnp.int32, (16,), 0)
    rconsts = [jnp.full((16,), r, jnp.int32) for r in range(R)]

    def x_off(h, sub):
        return h * BATCH + b0 + sub * CHK

    def e_off(h, sub):
        return h * (8 * BATCH) + b0 * 8 + sub * STAGE

    def start_gather(h, sub, p):
        pltpu.sync_copy(x_hbm.at[pl.ds(x_off(h, sub), CHK)], idx_v.at[p])
        pltpu.async_copy(t_hbm.at[idx_v.at[p]], rows_v.at[p], gsems[p])

    start_gather(h0, 0, 0)

    def outer(i, carry):
        h = h0 + i
        for p in range(2):
            c = 2 * i + p

            @pl.when(c + 1 < NCH)
            def _():
                nh = h0 + (c + 1) // 2
                start_gather(nh, (c + 1) % 2, 1 - p)

            pltpu.make_async_copy(t_hbm.at[pl.ds(0, CHK)], rows_v.at[p],
                                  gsems[p]).wait()

            @pl.when(c >= 2)
            def _():
                pltpu.make_async_copy(
                    stage_v.at[pl.ds(0, STAGE)],
                    e_hbm.at[pl.ds(0, STAGE)], osems[p]).wait()

            rv = rows_v.at[p]
            sbase = p * STAGE

            def dstep(j, cc):
                t0 = j * 16
                dst0 = sbase + (t0 // 128) * 1024 + t0 % 128
                for r in range(R):
                    vals = plsc.load_gather(rv, [t0 + lanes, rconsts[r]])
                    stage_v[pl.ds(dst0 + r * 128, 16)] = vals
                return cc

            lax.fori_loop(0, CHK // 16, dstep, 0)
            pltpu.async_copy(stage_v.at[pl.ds(sbase, STAGE)],
                             e_hbm.at[pl.ds(e_off(h, p), STAGE)], osems[p])
        return carry

    lax.fori_loop(0, HPW, outer, 0)
    for p in range(2):
        pltpu.make_async_copy(stage_v.at[pl.ds(0, STAGE)],
                              e_hbm.at[pl.ds(0, STAGE)], osems[p]).wait()


def _matmul_body(b_ref, e_ref, o_ref):
    b8 = b_ref[...]
    for bt in range(BATCH // 128):
        e = e_ref[pl.ds(bt * 8, 8), :]
        o_ref[0, :, pl.ds(bt * 128, 128)] = jnp.dot(
            b8, e, preferred_element_type=jnp.float32)


@jax.jit
def _run(x, A, B):
    x_flat = x.T.reshape(NTOK).astype(jnp.int32)
    a_flat = A.reshape(R * NUM_EMB)

    mesh = plsc.VectorSubcoreMesh(core_axis_name="c", subcore_axis_name="s")
    pack = pl.kernel(
        _pack_body,
        out_type=jax.ShapeDtypeStruct((8 * NUM_EMB,), jnp.float32),
        mesh=mesh,
        scratch_types=[
            pltpu.VMEM((R, VCH), jnp.float32),
            pltpu.VMEM((VCH * 8,), jnp.float32),
        ],
        compiler_params=_SC_PARAMS,
    )
    at_lin = pack(a_flat)
    at = at_lin.reshape(NUM_EMB, 8)

    gather = pl.kernel(
        _gather_body,
        out_type=jax.ShapeDtypeStruct((HIST * 8 * BATCH,), jnp.float32),
        mesh=mesh,
        scratch_types=[
            pltpu.VMEM((2, CHK), jnp.int32),
            pltpu.VMEM((2, CHK, 8), jnp.float32),
            pltpu.VMEM((2 * STAGE,), jnp.float32),
            pltpu.SemaphoreType.DMA,
            pltpu.SemaphoreType.DMA,
            pltpu.SemaphoreType.DMA,
            pltpu.SemaphoreType.DMA,
        ],
        compiler_params=_SC_PARAMS,
    )
    e_flat = gather(at, x_flat)
    e2 = e_flat.reshape(HIST * 8 * BATCH // 128, 128)

    B8 = jnp.concatenate([B * SCALING, jnp.zeros((DIM, 4), jnp.float32)],
                         axis=1)
    outT = pl.pallas_call(
        _matmul_body,
        grid=(HIST,),
        in_specs=[
            pl.BlockSpec((DIM, 8), lambda i: (0, 0)),
            pl.BlockSpec((BATCH // 16, 128), lambda i: (i, 0)),
        ],
        out_specs=pl.BlockSpec((1, DIM, BATCH), lambda i: (i, 0, 0)),
        out_shape=jax.ShapeDtypeStruct((HIST, DIM, BATCH), jnp.float32),
    )(B8, e2)
    return jnp.transpose(outT, (2, 0, 1))


def kernel(x, A, B):
    return _run(x, A, B)


# speedup vs baseline: 109.1385x; 1.2738x over previous
import jax
import jax.numpy as jnp
from jax import lax
from jax.experimental import pallas as pl
from jax.experimental.pallas import tpu as pltpu
from jax.experimental.pallas import tpu_sc as plsc

NUM_EMB = 1000000
DIM = 32
R = 4
SCALING = 1.0 / 4.0
BATCH = 16384
HIST = 200
NTOK = BATCH * HIST

_info = plsc.get_sparse_core_info()
NC, NS = _info.num_cores, _info.num_subcores
NW = NC * NS

_SC_PARAMS = pltpu.CompilerParams(use_tc_tiling_on_sc=False,
                                  needs_layout_passes=False)


VCH = 8000
NVCH = NUM_EMB // VCH
CPW = (NVCH + NW - 1) // NW


def _pack_body(a_hbm, at_hbm, planes_v, stage_v):
    wid = lax.axis_index("s") * NC + lax.axis_index("c")
    lanes = lax.broadcasted_iota(jnp.int32, (16,), 0)
    r_idx = lanes & 3
    l4 = lanes >> 2
    pat = l4 * 8 + r_idx
    zero = jnp.zeros((16,), jnp.float32)

    def zstep(i, c):
        for u in range(8):
            stage_v[pl.ds((i * 8 + u) * 16, 16)] = zero
        return c

    lax.fori_loop(0, VCH * 8 // 128, zstep, 0)

    def chunk(i, carry):
        c = wid + i * NW

        @pl.when(c < NVCH)
        def _():
            for r in range(R):
                pltpu.sync_copy(a_hbm.at[pl.ds(r * NUM_EMB + c * VCH, VCH)],
                                planes_v.at[r])

            def g_step(g, cc):
                for u in range(2):
                    gg = g * 2 + u
                    vals = plsc.load_gather(planes_v, [r_idx, gg * 4 + l4])
                    plsc.store_scatter(stage_v, [gg * 32 + pat], vals)
                return cc

            lax.fori_loop(0, VCH * 4 // 32, g_step, 0)
            pltpu.sync_copy(stage_v, at_hbm.at[pl.ds(c * VCH * 8, VCH * 8)])

        return carry

    lax.fori_loop(0, CPW, chunk, 0)


NHG = 4
NBG = NW // NHG
CHK = 2048
STAGE = CHK * 8
HSPAN = HIST // 2
HPW = HSPAN // NHG
NCH = HPW


def _make_gather_body(h_base):
    def _gather_body(t_hbm, x_hbm, e_hbm, idx_v, rows_v, stage_v,
                     gsem0, gsem1, osem0, osem1):
        wid = lax.axis_index("s") * NC + lax.axis_index("c")
        hg = wid >> 3
        bg = wid & 7
        b0 = bg * CHK
        gsems = (gsem0, gsem1)
        osems = (osem0, osem1)
        zero = jnp.zeros((16,), jnp.float32)

        def zstep(i, c):
            for u in range(8):
                stage_v[pl.ds((i * 8 + u) * 16, 16)] = zero
            return c

        lax.fori_loop(0, 2 * STAGE // 128, zstep, 0)

        lanes = lax.broadcasted_iota(jnp.int32, (16,), 0)
        rconsts = [jnp.full((16,), r, jnp.int32) for r in range(R)]

        def start_gather(c, p):
            h = h_base + hg * HPW + c
            pltpu.sync_copy(x_hbm.at[pl.ds(h * BATCH + b0, CHK)],
                            idx_v.at[p])
            pltpu.async_copy(t_hbm.at[idx_v.at[p]], rows_v.at[p], gsems[p])

        start_gather(0, 0)

        def outer(i, carry):
            for p in range(2):
                c = 2 * i + p

                @pl.when(c < NCH)
                def _():
                    @pl.when(c + 1 < NCH)
                    def _():
                        start_gather(c + 1, 1 - p)

                    pltpu.make_async_copy(t_hbm.at[pl.ds(0, CHK)],
                                          rows_v.at[p], gsems[p]).wait()

                    @pl.when(c >= 2)
                    def _():
                        pltpu.make_async_copy(
                            stage_v.at[pl.ds(0, STAGE)],
                            e_hbm.at[pl.ds(0, STAGE)], osems[p]).wait()

                    rv = rows_v.at[p]
                    sbase = p * STAGE

                    def dstep(j, cc):
                        for u in range(2):
                            t0 = (j * 2 + u) * 16
                            dst0 = sbase + (t0 // 128) * 1024 + t0 % 128
                            for r in range(R):
                                vals = plsc.load_gather(
                                    rv, [t0 + lanes, rconsts[r]])
                                stage_v[pl.ds(dst0 + r * 128, 16)] = vals
                        return cc

                    lax.fori_loop(0, CHK // 32, dstep, 0)
                    hl = hg * HPW + c
                    pltpu.async_copy(
                        stage_v.at[pl.ds(sbase, STAGE)],
                        e_hbm.at[pl.ds(hl * (8 * BATCH) + b0 * 8, STAGE)],
                        osems[p])
            return carry

        lax.fori_loop(0, (NCH + 1) // 2, outer, 0)
        for p in range(2):
            pltpu.make_async_copy(stage_v.at[pl.ds(0, STAGE)],
                                  e_hbm.at[pl.ds(0, STAGE)], osems[p]).wait()

    return _gather_body


def _matmul_body(b_ref, e_ref, o_ref):
    b8 = b_ref[...]
    for bt in range(BATCH // 128):
        e = e_ref[pl.ds(bt * 8, 8), :]
        o_ref[0, :, pl.ds(bt * 128, 128)] = jnp.dot(
            b8, e, preferred_element_type=jnp.float32)


def _matmul_body_aliased(b_ref, e_ref, full_ref, o_ref):
    _matmul_body(b_ref, e_ref, o_ref)


@jax.jit
def _run(x, A, B):
    x_flat = x.T.reshape(NTOK).astype(jnp.int32)
    a_flat = A.reshape(R * NUM_EMB)

    mesh = plsc.VectorSubcoreMesh(core_axis_name="c", subcore_axis_name="s")
    pack = pl.kernel(
        _pack_body,
        out_type=jax.ShapeDtypeStruct((8 * NUM_EMB,), jnp.float32),
        mesh=mesh,
        scratch_types=[
            pltpu.VMEM((R, VCH), jnp.float32),
            pltpu.VMEM((VCH * 8,), jnp.float32),
        ],
        compiler_params=_SC_PARAMS,
    )
    at = pack(a_flat).reshape(NUM_EMB, 8)

    def gather_half(h_base):
        g = pl.kernel(
            _make_gather_body(h_base),
            out_type=jax.ShapeDtypeStruct((HSPAN * 8 * BATCH,), jnp.float32),
            mesh=mesh,
            scratch_types=[
                pltpu.VMEM((2, CHK), jnp.int32),
                pltpu.VMEM((2, CHK, 8), jnp.float32),
                pltpu.VMEM((2 * STAGE,), jnp.float32),
                pltpu.SemaphoreType.DMA,
                pltpu.SemaphoreType.DMA,
                pltpu.SemaphoreType.DMA,
                pltpu.SemaphoreType.DMA,
            ],
            compiler_params=_SC_PARAMS,
        )
        return g(at, x_flat).reshape(HSPAN * 8 * BATCH // 128, 128)

    e_a = gather_half(0)
    e_b = gather_half(HSPAN)

    B8 = jnp.concatenate([B * SCALING, jnp.zeros((DIM, 4), jnp.float32)],
                         axis=1)
    half0 = pl.pallas_call(
        _matmul_body,
        grid=(HSPAN,),
        in_specs=[
            pl.BlockSpec((DIM, 8), lambda i: (0, 0)),
            pl.BlockSpec((BATCH // 16, 128), lambda i: (i, 0)),
        ],
        out_specs=pl.BlockSpec((1, DIM, BATCH), lambda i: (i, 0, 0)),
        out_shape=jax.ShapeDtypeStruct((HIST, DIM, BATCH), jnp.float32),
    )(B8, e_a)
    outT = pl.pallas_call(
        _matmul_body_aliased,
        grid=(HSPAN,),
        in_specs=[
            pl.BlockSpec((DIM, 8), lambda i: (0, 0)),
            pl.BlockSpec((BATCH // 16, 128), lambda i: (i, 0)),
            pl.BlockSpec(memory_space=pl.ANY),
        ],
        out_specs=pl.BlockSpec((1, DIM, BATCH), lambda i: (i + HSPAN, 0, 0)),
        out_shape=jax.ShapeDtypeStruct((HIST, DIM, BATCH), jnp.float32),
        input_output_aliases={2: 0},
    )(B8, e_b, half0)
    return jnp.transpose(outT, (2, 0, 1))


def kernel(x, A, B):
    return _run(x, A, B)


# speedup vs baseline: 111.8012x; 1.0244x over previous
import jax
import jax.numpy as jnp
from jax import lax
from jax.experimental import pallas as pl
from jax.experimental.pallas import tpu as pltpu
from jax.experimental.pallas import tpu_sc as plsc

NUM_EMB = 1000000
DIM = 32
R = 4
SCALING = 1.0 / 4.0
BATCH = 16384
HIST = 200
NTOK = BATCH * HIST

_info = plsc.get_sparse_core_info()
NC, NS = _info.num_cores, _info.num_subcores
NW = NC * NS

_SC_PARAMS = pltpu.CompilerParams(use_tc_tiling_on_sc=False,
                                  needs_layout_passes=False)


VCH = 8000
NVCH = NUM_EMB // VCH
CPW = (NVCH + NW - 1) // NW


def _pack_body(a_hbm, at_hbm, planes_v, stage_v):
    wid = lax.axis_index("s") * NC + lax.axis_index("c")
    lanes = lax.broadcasted_iota(jnp.int32, (16,), 0)
    r_idx = lanes & 3
    l4 = lanes >> 2
    pat = l4 * 8 + r_idx
    zero = jnp.zeros((16,), jnp.float32)

    def zstep(i, c):
        for u in range(8):
            stage_v[pl.ds((i * 8 + u) * 16, 16)] = zero
        return c

    lax.fori_loop(0, VCH * 8 // 128, zstep, 0)

    def chunk(i, carry):
        c = wid + i * NW

        @pl.when(c < NVCH)
        def _():
            for r in range(R):
                pltpu.sync_copy(a_hbm.at[pl.ds(r * NUM_EMB + c * VCH, VCH)],
                                planes_v.at[r])

            def g_step(g, cc):
                for u in range(2):
                    gg = g * 2 + u
                    vals = plsc.load_gather(planes_v, [r_idx, gg * 4 + l4])
                    plsc.store_scatter(stage_v, [gg * 32 + pat], vals)
                return cc

            lax.fori_loop(0, VCH * 4 // 32, g_step, 0)
            pltpu.sync_copy(stage_v, at_hbm.at[pl.ds(c * VCH * 8, VCH * 8)])

        return carry

    lax.fori_loop(0, CPW, chunk, 0)


HPARTS = 4
NHG = 2
NBG = NW // NHG
CHK = 1024
STAGE = CHK * 8
HSPAN = HIST // HPARTS
HPW = HSPAN // NHG
NCH = HPW


def _make_gather_body(h_base):
    def _gather_body(t_hbm, x_hbm, e_hbm, idx_v, rows_v, stage_v,
                     gsem0, gsem1, osem0, osem1):
        wid = lax.axis_index("s") * NC + lax.axis_index("c")
        hg = wid >> 4
        bg = wid & 15
        b0 = bg * CHK
        gsems = (gsem0, gsem1)
        osems = (osem0, osem1)
        zero = jnp.zeros((16,), jnp.float32)

        def zstep(i, c):
            for u in range(8):
                stage_v[pl.ds((i * 8 + u) * 16, 16)] = zero
            return c

        lax.fori_loop(0, 2 * STAGE // 128, zstep, 0)

        lanes = lax.broadcasted_iota(jnp.int32, (16,), 0)
        rconsts = [jnp.full((16,), r, jnp.int32) for r in range(R)]

        def start_gather(c, p):
            h = h_base + hg * HPW + c
            pltpu.sync_copy(x_hbm.at[pl.ds(h * BATCH + b0, CHK)],
                            idx_v.at[p])
            pltpu.async_copy(t_hbm.at[idx_v.at[p]], rows_v.at[p], gsems[p])

        start_gather(0, 0)

        def outer(i, carry):
            for p in range(2):
                c = 2 * i + p

                @pl.when(c < NCH)
                def _():
                    @pl.when(c + 1 < NCH)
                    def _():
                        start_gather(c + 1, 1 - p)

                    pltpu.make_async_copy(t_hbm.at[pl.ds(0, CHK)],
                                          rows_v.at[p], gsems[p]).wait()

                    @pl.when(c >= 2)
                    def _():
                        pltpu.make_async_copy(
                            stage_v.at[pl.ds(0, STAGE)],
                            e_hbm.at[pl.ds(0, STAGE)], osems[p]).wait()

                    rv = rows_v.at[p]
                    sbase = p * STAGE

                    def dstep(j, cc):
                        for u in range(2):
                            t0 = (j * 2 + u) * 16
                            dst0 = sbase + (t0 // 128) * 1024 + t0 % 128
                            for r in range(R):
                                vals = plsc.load_gather(
                                    rv, [t0 + lanes, rconsts[r]])
                                stage_v[pl.ds(dst0 + r * 128, 16)] = vals
                        return cc

                    lax.fori_loop(0, CHK // 32, dstep, 0)
                    hl = hg * HPW + c
                    pltpu.async_copy(
                        stage_v.at[pl.ds(sbase, STAGE)],
                        e_hbm.at[pl.ds(hl * (8 * BATCH) + b0 * 8, STAGE)],
                        osems[p])
            return carry

        lax.fori_loop(0, (NCH + 1) // 2, outer, 0)
        for p in range(2):
            pltpu.make_async_copy(stage_v.at[pl.ds(0, STAGE)],
                                  e_hbm.at[pl.ds(0, STAGE)], osems[p]).wait()

    return _gather_body


def _matmul_body(b_ref, e_ref, o_ref):
    b8 = b_ref[...]
    for bt in range(BATCH // 128):
        e = e_ref[pl.ds(bt * 8, 8), :]
        o_ref[0, :, pl.ds(bt * 128, 128)] = jnp.dot(
            b8, e, preferred_element_type=jnp.float32)


def _matmul_body_aliased(b_ref, e_ref, full_ref, o_ref):
    _matmul_body(b_ref, e_ref, o_ref)


@jax.jit
def _run(x, A, B):
    x_flat = x.T.reshape(NTOK).astype(jnp.int32)
    a_flat = A.reshape(R * NUM_EMB)

    mesh = plsc.VectorSubcoreMesh(core_axis_name="c", subcore_axis_name="s")
    pack = pl.kernel(
        _pack_body,
        out_type=jax.ShapeDtypeStruct((8 * NUM_EMB,), jnp.float32),
        mesh=mesh,
        scratch_types=[
            pltpu.VMEM((R, VCH), jnp.float32),
            pltpu.VMEM((VCH * 8,), jnp.float32),
        ],
        compiler_params=_SC_PARAMS,
    )
    at = pack(a_flat).reshape(NUM_EMB, 8)

    def gather_half(h_base):
        g = pl.kernel(
            _make_gather_body(h_base),
            out_type=jax.ShapeDtypeStruct((HSPAN * 8 * BATCH,), jnp.float32),
            mesh=mesh,
            scratch_types=[
                pltpu.VMEM((2, CHK), jnp.int32),
                pltpu.VMEM((2, CHK, 8), jnp.float32),
                pltpu.VMEM((2 * STAGE,), jnp.float32),
                pltpu.SemaphoreType.DMA,
                pltpu.SemaphoreType.DMA,
                pltpu.SemaphoreType.DMA,
                pltpu.SemaphoreType.DMA,
            ],
            compiler_params=_SC_PARAMS,
        )
        return g(at, x_flat).reshape(HSPAN * 8 * BATCH // 128, 128)

    e_parts = [gather_half(q * HSPAN) for q in range(HPARTS)]

    B8 = jnp.concatenate([B * SCALING, jnp.zeros((DIM, 4), jnp.float32)],
                         axis=1)

    def matmul_part(q, e_q, prev):
        def omap(i, q=q):
            return (i + q * HSPAN, 0, 0)

        common = dict(
            grid=(HSPAN,),
            out_specs=pl.BlockSpec((1, DIM, BATCH), omap),
            out_shape=jax.ShapeDtypeStruct((HIST, DIM, BATCH), jnp.float32),
        )
        especs = [
            pl.BlockSpec((DIM, 8), lambda i: (0, 0)),
            pl.BlockSpec((BATCH // 16, 128), lambda i: (i, 0)),
        ]
        if prev is None:
            return pl.pallas_call(_matmul_body, in_specs=especs,
                                  **common)(B8, e_q)
        return pl.pallas_call(
            _matmul_body_aliased,
            in_specs=especs + [pl.BlockSpec(memory_space=pl.ANY)],
            input_output_aliases={2: 0},
            **common)(B8, e_q, prev)

    outT = None
    for q in range(HPARTS):
        outT = matmul_part(q, e_parts[q], outT)
    return jnp.transpose(outT, (2, 0, 1))


def kernel(x, A, B):
    return _run(x, A, B)


# speedup vs baseline: 114.2713x; 1.0221x over previous
import jax
import jax.numpy as jnp
from jax import lax
from jax.experimental import pallas as pl
from jax.experimental.pallas import tpu as pltpu
from jax.experimental.pallas import tpu_sc as plsc

NUM_EMB = 1000000
DIM = 32
R = 4
SCALING = 1.0 / 4.0
BATCH = 16384
HIST = 200
NTOK = BATCH * HIST

_info = plsc.get_sparse_core_info()
NC, NS = _info.num_cores, _info.num_subcores
NW = NC * NS

_SC_PARAMS = pltpu.CompilerParams(use_tc_tiling_on_sc=False,
                                  needs_layout_passes=False)


VCH = 8000
NVCH = NUM_EMB // VCH
CPW = (NVCH + NW - 1) // NW


def _pack_body(a_hbm, at_hbm, planes_v, stage_v):
    wid = lax.axis_index("s") * NC + lax.axis_index("c")
    lanes = lax.broadcasted_iota(jnp.int32, (16,), 0)
    r_idx = lanes & 3
    l4 = lanes >> 2
    pat = l4 * 8 + r_idx
    zero = jnp.zeros((16,), jnp.float32)

    def zstep(i, c):
        for u in range(8):
            stage_v[pl.ds((i * 8 + u) * 16, 16)] = zero
        return c

    lax.fori_loop(0, VCH * 8 // 128, zstep, 0)

    def chunk(i, carry):
        c = wid + i * NW

        @pl.when(c < NVCH)
        def _():
            for r in range(R):
                pltpu.sync_copy(a_hbm.at[pl.ds(r * NUM_EMB + c * VCH, VCH)],
                                planes_v.at[r])

            def g_step(g, cc):
                for u in range(2):
                    gg = g * 2 + u
                    vals = plsc.load_gather(planes_v, [r_idx, gg * 4 + l4])
                    plsc.store_scatter(stage_v, [gg * 32 + pat], vals)
                return cc

            lax.fori_loop(0, VCH * 4 // 32, g_step, 0)
            pltpu.sync_copy(stage_v, at_hbm.at[pl.ds(c * VCH * 8, VCH * 8)])

        return carry

    lax.fori_loop(0, CPW, chunk, 0)


HPARTS = 4
NHG = 2
NBG = NW // NHG
CHK = 1024
STAGE = CHK * 8
HSPAN = HIST // HPARTS
HPW = HSPAN // NHG
NCH = HPW


def _make_gather_body(h_base):
    def _gather_body(t_hbm, x_hbm, e_hbm, idx_v, rows_v, stage_v,
                     gsem0, gsem1, osem0, osem1):
        wid = lax.axis_index("s") * NC + lax.axis_index("c")
        hg = wid >> 4
        bg = wid & 15
        b0 = bg * CHK
        gsems = (gsem0, gsem1)
        osems = (osem0, osem1)
        zero = jnp.zeros((16,), jnp.float32)

        def zstep(i, c):
            for u in range(8):
                stage_v[pl.ds((i * 8 + u) * 16, 16)] = zero
            return c

        lax.fori_loop(0, 2 * STAGE // 128, zstep, 0)

        lanes = lax.broadcasted_iota(jnp.int32, (16,), 0)
        rconsts = [jnp.full((16,), r, jnp.int32) for r in range(R)]

        pltpu.sync_copy(
            x_hbm.at[pl.ds(h_base + hg * HPW, HPW), pl.ds(b0, CHK)], idx_v)

        def start_gather(c, p):
            pltpu.async_copy(t_hbm.at[idx_v.at[c]], rows_v.at[p], gsems[p])

        start_gather(0, 0)

        def outer(i, carry):
            for p in range(2):
                c = 2 * i + p

                @pl.when(c < NCH)
                def _():
                    @pl.when(c + 1 < NCH)
                    def _():
                        start_gather(c + 1, 1 - p)

                    pltpu.make_async_copy(t_hbm.at[pl.ds(0, CHK)],
                                          rows_v.at[p], gsems[p]).wait()

                    @pl.when(c >= 2)
                    def _():
                        pltpu.make_async_copy(
                            stage_v.at[pl.ds(0, STAGE)],
                            e_hbm.at[pl.ds(0, STAGE)], osems[p]).wait()

                    rv = rows_v.at[p]
                    sbase = p * STAGE

                    def dstep(j, cc):
                        for u in range(2):
                            t0 = (j * 2 + u) * 16
                            dst0 = sbase + (t0 // 128) * 1024 + t0 % 128
                            for r in range(R):
                                vals = plsc.load_gather(
                                    rv, [t0 + lanes, rconsts[r]])
                                stage_v[pl.ds(dst0 + r * 128, 16)] = vals
                        return cc

                    lax.fori_loop(0, CHK // 32, dstep, 0)
                    hl = hg * HPW + c
                    pltpu.async_copy(
                        stage_v.at[pl.ds(sbase, STAGE)],
                        e_hbm.at[pl.ds(hl * (8 * BATCH) + b0 * 8, STAGE)],
                        osems[p])
            return carry

        lax.fori_loop(0, (NCH + 1) // 2, outer, 0)
        for p in range(2):
            pltpu.make_async_copy(stage_v.at[pl.ds(0, STAGE)],
                                  e_hbm.at[pl.ds(0, STAGE)], osems[p]).wait()

    return _gather_body


def _matmul_body(b_ref, e_ref, o_ref):
    b8 = b_ref[...]
    for bt in range(BATCH // 128):
        e = e_ref[pl.ds(bt * 8, 8), :]
        o_ref[0, :, pl.ds(bt * 128, 128)] = jnp.dot(
            b8, e, preferred_element_type=jnp.float32)


def _matmul_body_aliased(b_ref, e_ref, full_ref, o_ref):
    _matmul_body(b_ref, e_ref, o_ref)


@jax.jit
def _run(x, A, B):
    x2 = x.T.reshape(NTOK).astype(jnp.int32).reshape(HIST, BATCH)
    a_flat = A.reshape(R * NUM_EMB)

    mesh = plsc.VectorSubcoreMesh(core_axis_name="c", subcore_axis_name="s")
    pack = pl.kernel(
        _pack_body,
        out_type=jax.ShapeDtypeStruct((8 * NUM_EMB,), jnp.float32),
        mesh=mesh,
        scratch_types=[
            pltpu.VMEM((R, VCH), jnp.float32),
            pltpu.VMEM((VCH * 8,), jnp.float32),
        ],
        compiler_params=_SC_PARAMS,
    )
    at = pack(a_flat).reshape(NUM_EMB, 8)

    def gather_half(h_base):
        g = pl.kernel(
            _make_gather_body(h_base),
            out_type=jax.ShapeDtypeStruct((HSPAN * 8 * BATCH,), jnp.float32),
            mesh=mesh,
            scratch_types=[
                pltpu.VMEM((NCH, CHK), jnp.int32),
                pltpu.VMEM((2, CHK, 8), jnp.float32),
                pltpu.VMEM((2 * STAGE,), jnp.float32),
                pltpu.SemaphoreType.DMA,
                pltpu.SemaphoreType.DMA,
                pltpu.SemaphoreType.DMA,
                pltpu.SemaphoreType.DMA,
            ],
            compiler_params=_SC_PARAMS,
        )
        return g(at, x2).reshape(HSPAN * 8 * BATCH // 128, 128)

    e_parts = [gather_half(q * HSPAN) for q in range(HPARTS)]

    B8 = jnp.concatenate([B * SCALING, jnp.zeros((DIM, 4), jnp.float32)],
                         axis=1)

    def matmul_part(q, e_q, prev):
        def omap(i, q=q):
            return (i + q * HSPAN, 0, 0)

        common = dict(
            grid=(HSPAN,),
            out_specs=pl.BlockSpec((1, DIM, BATCH), omap),
            out_shape=jax.ShapeDtypeStruct((HIST, DIM, BATCH), jnp.float32),
        )
        especs = [
            pl.BlockSpec((DIM, 8), lambda i: (0, 0)),
            pl.BlockSpec((BATCH // 16, 128), lambda i: (i, 0)),
        ]
        if prev is None:
            return pl.pallas_call(_matmul_body, in_specs=especs,
                                  **common)(B8, e_q)
        return pl.pallas_call(
            _matmul_body_aliased,
            in_specs=especs + [pl.BlockSpec(memory_space=pl.ANY)],
            input_output_aliases={2: 0},
            **common)(B8, e_q, prev)

    outT = None
    for q in range(HPARTS):
        outT = matmul_part(q, e_parts[q], outT)
    return jnp.transpose(outT, (2, 0, 1))


def kernel(x, A, B):
    return _run(x, A, B)


# speedup vs baseline: 129.4201x; 1.1326x over previous
import jax
import jax.numpy as jnp
from jax import lax
from jax.experimental import pallas as pl
from jax.experimental.pallas import tpu as pltpu
from jax.experimental.pallas import tpu_sc as plsc

NUM_EMB = 1000000
DIM = 32
R = 4
SCALING = 1.0 / 4.0
BATCH = 16384
HIST = 200
NTOK = BATCH * HIST

_info = plsc.get_sparse_core_info()
NC, NS = _info.num_cores, _info.num_subcores
NW = NC * NS

_SC_PARAMS = pltpu.CompilerParams(use_tc_tiling_on_sc=False,
                                  needs_layout_passes=False)


VCH = 8000
NVCH = NUM_EMB // VCH
CPW = (NVCH + NW - 1) // NW


def _pack_body(a_hbm, at_hbm, planes_v, stage_v, psem):
    wid = lax.axis_index("s") * NC + lax.axis_index("c")
    lanes = lax.broadcasted_iota(jnp.int32, (16,), 0)
    r_idx = lanes & 3
    l4 = lanes >> 2
    pat = l4 * 8 + r_idx
    zero = jnp.zeros((16,), jnp.float32)

    def zstep(i, c):
        for u in range(8):
            stage_v[pl.ds((i * 8 + u) * 16, 16)] = zero
        return c

    lax.fori_loop(0, VCH * 8 // 128, zstep, 0)

    def chunk(i, carry):
        c = wid + i * NW

        @pl.when(c < NVCH)
        def _():
            for r in range(R):
                pltpu.async_copy(a_hbm.at[pl.ds(r * NUM_EMB + c * VCH, VCH)],
                                 planes_v.at[r], psem)
            for r in range(R):
                pltpu.make_async_copy(a_hbm.at[pl.ds(0, VCH)],
                                      planes_v.at[r], psem).wait()

            def g_step(g, cc):
                for u in range(4):
                    gg = g * 4 + u
                    vals = plsc.load_gather(planes_v, [r_idx, gg * 4 + l4])
                    plsc.store_scatter(stage_v, [gg * 32 + pat], vals)
                return cc

            lax.fori_loop(0, VCH * 4 // 64, g_step, 0)
            pltpu.sync_copy(stage_v, at_hbm.at[pl.ds(c * VCH * 8, VCH * 8)])

        return carry

    lax.fori_loop(0, CPW, chunk, 0)


HPARTS = 4
NHG = 2
NBG = NW // NHG
CHK = 1024
STAGE = CHK * 8
HSPAN = HIST // HPARTS
HPW = HSPAN // NHG
NCH = HPW


def _make_gather_body(h_base):
    def _gather_body(t_hbm, x_hbm, e_hbm, idx_v, rows_v, stage_v,
                     gsem0, gsem1, osem0, osem1):
        wid = lax.axis_index("s") * NC + lax.axis_index("c")
        hg = wid >> 4
        bg = wid & 15
        b0 = bg * CHK
        gsems = (gsem0, gsem1)
        osems = (osem0, osem1)
        zero = jnp.zeros((16,), jnp.float32)

        def zstep(i, c):
            for u in range(8):
                stage_v[pl.ds((i * 8 + u) * 16, 16)] = zero
            return c

        lax.fori_loop(0, 2 * STAGE // 128, zstep, 0)

        lanes = lax.broadcasted_iota(jnp.int32, (16,), 0)
        rconsts = [jnp.full((16,), r, jnp.int32) for r in range(R)]

        pltpu.sync_copy(
            x_hbm.at[pl.ds(h_base + hg * HPW, HPW), pl.ds(b0, CHK)], idx_v)

        def start_gather(c, p):
            pltpu.async_copy(t_hbm.at[idx_v.at[c]], rows_v.at[p], gsems[p])

        start_gather(0, 0)

        def outer(i, carry):
            for p in range(2):
                c = 2 * i + p

                @pl.when(c < NCH)
                def _():
                    @pl.when(c + 1 < NCH)
                    def _():
                        start_gather(c + 1, 1 - p)

                    pltpu.make_async_copy(t_hbm.at[pl.ds(0, CHK)],
                                          rows_v.at[p], gsems[p]).wait()

                    @pl.when(c >= 2)
                    def _():
                        pltpu.make_async_copy(
                            stage_v.at[pl.ds(0, STAGE)],
                            e_hbm.at[pl.ds(0, STAGE)], osems[p]).wait()

                    rv = rows_v.at[p]
                    sbase = p * STAGE

                    def dstep(j, cc):
                        for u in range(2):
                            t0 = (j * 2 + u) * 16
                            dst0 = sbase + (t0 // 128) * 1024 + t0 % 128
                            for r in range(R):
                                vals = plsc.load_gather(
                                    rv, [t0 + lanes, rconsts[r]])
                                stage_v[pl.ds(dst0 + r * 128, 16)] = vals
                        return cc

                    lax.fori_loop(0, CHK // 32, dstep, 0)
                    hl = hg * HPW + c
                    pltpu.async_copy(
                        stage_v.at[pl.ds(sbase, STAGE)],
                        e_hbm.at[pl.ds(hl * (8 * BATCH) + b0 * 8, STAGE)],
                        osems[p])
            return carry

        lax.fori_loop(0, (NCH + 1) // 2, outer, 0)
        for p in range(2):
            pltpu.make_async_copy(stage_v.at[pl.ds(0, STAGE)],
                                  e_hbm.at[pl.ds(0, STAGE)], osems[p]).wait()

    return _gather_body


def _matmul_body(b_ref, e_ref, o_ref):
    b8 = b_ref[...]
    for bt in range(BATCH // 128):
        e = e_ref[pl.ds(bt * 8, 8), :]
        o_ref[0, :, pl.ds(bt * 128, 128)] = jnp.dot(
            b8, e, preferred_element_type=jnp.float32)


def _matmul_body_aliased(b_ref, e_ref, full_ref, o_ref):
    _matmul_body(b_ref, e_ref, o_ref)


@jax.jit
def _run(x, A, B):
    x2 = x.T.reshape(NTOK).astype(jnp.int32).reshape(HIST, BATCH)
    a_flat = A.reshape(R * NUM_EMB)

    mesh = plsc.VectorSubcoreMesh(core_axis_name="c", subcore_axis_name="s")
    pack = pl.kernel(
        _pack_body,
        out_type=jax.ShapeDtypeStruct((8 * NUM_EMB,), jnp.float32),
        mesh=mesh,
        scratch_types=[
            pltpu.VMEM((R, VCH), jnp.float32),
            pltpu.VMEM((VCH * 8,), jnp.float32),
            pltpu.SemaphoreType.DMA,
        ],
        compiler_params=_SC_PARAMS,
    )
    at = pack(a_flat).reshape(NUM_EMB, 8)

    def gather_half(h_base):
        g = pl.kernel(
            _make_gather_body(h_base),
            out_type=jax.ShapeDtypeStruct((HSPAN * 8 * BATCH,), jnp.float32),
            mesh=mesh,
            scratch_types=[
                pltpu.VMEM((NCH, CHK), jnp.int32),
                pltpu.VMEM((2, CHK, 8), jnp.float32),
                pltpu.VMEM((2 * STAGE,), jnp.float32),
                pltpu.SemaphoreType.DMA,
                pltpu.SemaphoreType.DMA,
                pltpu.SemaphoreType.DMA,
                pltpu.SemaphoreType.DMA,
            ],
            compiler_params=_SC_PARAMS,
        )
        return g(at, x2).reshape(HSPAN * 8 * BATCH // 128, 128)

    e_parts = [gather_half(q * HSPAN) for q in range(HPARTS)]

    B8 = jnp.concatenate([B * SCALING, jnp.zeros((DIM, 4), jnp.float32)],
                         axis=1)

    def matmul_part(q, e_q, prev):
        def omap(i, q=q):
            return (i + q * HSPAN, 0, 0)

        common = dict(
            grid=(HSPAN,),
            out_specs=pl.BlockSpec((1, DIM, BATCH), omap),
            out_shape=jax.ShapeDtypeStruct((HIST, DIM, BATCH), jnp.float32),
        )
        especs = [
            pl.BlockSpec((DIM, 8), lambda i: (0, 0)),
            pl.BlockSpec((BATCH // 16, 128), lambda i: (i, 0)),
        ]
        if prev is None:
            return pl.pallas_call(_matmul_body, in_specs=especs,
                                  **common)(B8, e_q)
        return pl.pallas_call(
            _matmul_body_aliased,
            in_specs=especs + [pl.BlockSpec(memory_space=pl.ANY)],
            input_output_aliases={2: 0},
            **common)(B8, e_q, prev)

    outT = None
    for q in range(HPARTS):
        outT = matmul_part(q, e_parts[q], outT)
    return jnp.transpose(outT, (2, 0, 1))


def kernel(x, A, B):
    return _run(x, A, B)
